# baseline (speedup 1.0000x reference)
import jax
import jax.numpy as jnp
from jax.experimental import pallas as pl
from jax.experimental.pallas import tpu as pltpu

_L = 256
_TAIL_CHUNKS = 2


def _body(pay_ref, pert_ref, emb_hbm, out_hbm, vin, vout, sem_tail, sem_in, sem_out):
    b, s, d = emb_hbm.shape
    tail = s - _L
    chunk = tail // _TAIL_CHUNKS
    tails = []
    for bi in range(b):
        for c in range(_TAIL_CHUNKS):
            lo = _L + c * chunk
            cp = pltpu.make_async_copy(
                emb_hbm.at[bi, pl.ds(lo, chunk), :],
                out_hbm.at[bi, pl.ds(lo, chunk), :],
                sem_tail.at[bi * _TAIL_CHUNKS + c],
            )
            cp.start()
            tails.append(cp)

    head_in = pltpu.make_async_copy(
        emb_hbm.at[:, pl.ds(0, _L), :], vin, sem_in
    )
    head_in.start()
    head_in.wait()
    mask = (pay_ref[...] == 1).astype(vout.dtype)
    vout[...] = vin[...] + (pert_ref[...] * mask)[None]
    head_out = pltpu.make_async_copy(
        vout, out_hbm.at[:, pl.ds(0, _L), :], sem_out
    )
    head_out.start()
    head_out.wait()
    for cp in tails:
        cp.wait()


def kernel(embedded_input, watermark_payload, perturbation_vectors):
    b, s, d = embedded_input.shape
    l = perturbation_vectors.shape[0]
    pay2d = watermark_payload.reshape(l, 1)
    return pl.pallas_call(
        _body,
        in_specs=[
            pl.BlockSpec((l, 1), lambda: (0, 0)),
            pl.BlockSpec((l, d), lambda: (0, 0)),
            pl.BlockSpec(memory_space=pltpu.MemorySpace.HBM),
        ],
        out_specs=pl.BlockSpec(memory_space=pltpu.MemorySpace.HBM),
        out_shape=jax.ShapeDtypeStruct((b, s, d), embedded_input.dtype),
        scratch_shapes=[
            pltpu.VMEM((b, _L, d), embedded_input.dtype),
            pltpu.VMEM((b, _L, d), embedded_input.dtype),
            pltpu.SemaphoreType.DMA((b * _TAIL_CHUNKS,)),
            pltpu.SemaphoreType.DMA,
            pltpu.SemaphoreType.DMA,
        ],
    )(pay2d, perturbation_vectors, embedded_input)


# speedup vs baseline: 45.0919x; 45.0919x over previous
import jax
import jax.numpy as jnp
from jax.experimental import pallas as pl
from jax.experimental.pallas import tpu as pltpu

_BLK = 1024
_L = 256


def _body(pay_ref, pert_ref, emb_ref, out_ref):
    j = pl.program_id(1)

    @pl.when(j == 0)
    def _():
        mask = (pay_ref[...] == 1).astype(out_ref.dtype)
        out_ref[0, :_L, :] = emb_ref[0, :_L, :] + pert_ref[...] * mask
        out_ref[0, _L:, :] = emb_ref[0, _L:, :]

    @pl.when(j != 0)
    def _():
        out_ref[...] = emb_ref[...]


def kernel(embedded_input, watermark_payload, perturbation_vectors):
    b, s, d = embedded_input.shape
    l = perturbation_vectors.shape[0]
    pay2d = watermark_payload.reshape(l, 1)
    return pl.pallas_call(
        _body,
        grid=(b, s // _BLK),
        in_specs=[
            pl.BlockSpec((l, 1), lambda bi, j: (0, 0)),
            pl.BlockSpec((l, d), lambda bi, j: (0, 0)),
            pl.BlockSpec((1, _BLK, d), lambda bi, j: (bi, j, 0)),
        ],
        out_specs=pl.BlockSpec((1, _BLK, d), lambda bi, j: (bi, j, 0)),
        out_shape=jax.ShapeDtypeStruct((b, s, d), embedded_input.dtype),
    )(pay2d, perturbation_vectors, embedded_input)
